# baseline (speedup 1.0000x reference)
import functools

import jax
import jax.numpy as jnp
from jax import lax
from jax.experimental import pallas as pl
from jax.experimental.pallas import tpu as pltpu
from jax.experimental.pallas import tpu_sc as plsc

D = 1024
HALF = 512
VOCAB = 194
TBL_N = VOCAB * HALF
NC = 2
NS = 16
NW = NC * NS
B_TOT = 1024 * 200
B_PER_W = B_TOT // NW
G = 16
NGROUP = B_PER_W // G
BUF_N = G * HALF


def _sc_gather(ids_flat, table_halves):
    mesh = plsc.VectorSubcoreMesh(core_axis_name="c", subcore_axis_name="s")

    @functools.partial(
        pl.kernel,
        mesh=mesh,
        compiler_params=pltpu.CompilerParams(
            use_tc_tiling_on_sc=False, needs_layout_passes=False
        ),
        out_type=jax.ShapeDtypeStruct((B_TOT * D,), jnp.float32),
        scratch_types=[
            pltpu.VMEM((B_PER_W,), jnp.int32),
            pltpu.VMEM((TBL_N,), jnp.float32),
            pltpu.VMEM((BUF_N,), jnp.float32),
            pltpu.VMEM((BUF_N,), jnp.float32),
            pltpu.SemaphoreType.DMA,
            pltpu.SemaphoreType.DMA,
        ],
    )
    def k(ids_hbm, tbl_hbm, out_hbm, idx_v, tbl_v, buf0, buf1, s0, s1):
        wid = lax.axis_index("s") * NC + lax.axis_index("c")
        base = pl.multiple_of(wid * B_PER_W, 8)
        pltpu.sync_copy(ids_hbm.at[pl.ds(base, B_PER_W)], idx_v)

        bufs = (buf0, buf1)
        ssems = (s0, s1)
        lanes = lax.iota(jnp.int32, G) * HALF

        def s_drain(b):
            pltpu.make_async_copy(
                out_hbm.at[pl.ds(0, BUF_N)], bufs[b], ssems[b]
            ).wait()

        def one_pass(p, carry):
            pltpu.sync_copy(
                tbl_hbm.at[pl.ds(pl.multiple_of(p * TBL_N, 8), TBL_N)], tbl_v
            )

            def pair(pr, c2):
                for b in range(2):
                    g = pr * 2 + b
                    t = p * NGROUP + g

                    @pl.when(t >= 2)
                    def _():
                        s_drain(b)

                    off = pl.multiple_of(g * G, 8)
                    gbase = idx_v[pl.ds(off, G)] * HALF
                    delta = lanes - gbase
                    CB = 8
                    prev = None
                    for cb in range(0, HALF, CB):
                        cur = []
                        for j in range(CB):
                            addr = gbase + (cb + j)
                            cur.append((addr, plsc.load_gather(tbl_v, [addr])))
                            if prev is not None:
                                paddr, pval = prev[j]
                                plsc.store_scatter(bufs[b], [paddr + delta], pval)
                        prev = cur
                    for paddr, pval in prev:
                        plsc.store_scatter(bufs[b], [paddr + delta], pval)
                    row0 = base + g * G
                    for r in range(G):
                        flat = pl.multiple_of((row0 + r) * D + p * HALF, 8)
                        pltpu.async_copy(
                            bufs[b].at[pl.ds(r * HALF, HALF)],
                            out_hbm.at[pl.ds(flat, HALF)],
                            ssems[b],
                        )
                return c2

            lax.fori_loop(0, NGROUP // 2, pair, 0)
            return carry

        lax.fori_loop(0, D // HALF, one_pass, 0)
        s_drain(0)
        s_drain(1)

    return k(ids_flat, table_halves)


def kernel(phoneme_ids, table):
    ids_flat = phoneme_ids.reshape(-1)
    table_halves = table.reshape(VOCAB, 2, HALF).transpose(1, 0, 2).reshape(-1)
    out = _sc_gather(ids_flat, table_halves)
    return out.reshape(phoneme_ids.shape[0], phoneme_ids.shape[1], D)


# speedup vs baseline: 3.3836x; 3.3836x over previous
import functools

import jax
import jax.numpy as jnp
from jax import lax
from jax.experimental import pallas as pl
from jax.experimental.pallas import tpu as pltpu
from jax.experimental.pallas import tpu_sc as plsc

D = 1024
HALF = 512
VOCAB = 194
TBL_N = VOCAB * HALF
NC = 2
NS = 16
NW = NC * NS
B_TOT = 1024 * 200
B_PER_W = B_TOT // NW
G = 16
NGROUP = B_PER_W // G
BUF_N = G * HALF


def _sc_gather(ids_flat, table_halves):
    mesh = plsc.VectorSubcoreMesh(core_axis_name="c", subcore_axis_name="s")

    @functools.partial(
        pl.kernel,
        mesh=mesh,
        compiler_params=pltpu.CompilerParams(
            use_tc_tiling_on_sc=False, needs_layout_passes=False
        ),
        out_type=jax.ShapeDtypeStruct((B_TOT * D,), jnp.float32),
        scratch_types=[
            pltpu.VMEM((B_PER_W,), jnp.int32),
            pltpu.VMEM((TBL_N,), jnp.float32),
            pltpu.VMEM((BUF_N,), jnp.float32),
            pltpu.VMEM((BUF_N,), jnp.float32),
            pltpu.SemaphoreType.DMA,
            pltpu.SemaphoreType.DMA,
        ],
    )
    def k(ids_hbm, tbl_hbm, out_hbm, idx_v, tbl_v, buf0, buf1, s0, s1):
        wid = lax.axis_index("s") * NC + lax.axis_index("c")
        base = pl.multiple_of(wid * B_PER_W, 8)
        pltpu.sync_copy(ids_hbm.at[pl.ds(base, B_PER_W)], idx_v)

        bufs = (buf0, buf1)
        ssems = (s0, s1)
        iota = lax.iota(jnp.int32, G)
        lanes = iota * HALF
        skew = iota * (HALF + 1)

        def s_drain(b):
            pltpu.make_async_copy(
                out_hbm.at[pl.ds(0, BUF_N)], bufs[b], ssems[b]
            ).wait()

        def one_pass(p, carry):
            pltpu.sync_copy(
                tbl_hbm.at[pl.ds(pl.multiple_of(p * TBL_N, 8), TBL_N)], tbl_v
            )

            def pair(pr, c2):
                for b in range(2):
                    g = pr * 2 + b
                    t = p * NGROUP + g

                    @pl.when(t >= 2)
                    def _():
                        s_drain(b)

                    off = pl.multiple_of(g * G, 8)
                    gbase = idx_v[pl.ds(off, G)] * HALF
                    gbs = [gbase | (iota ^ pb) for pb in range(16)]
                    sbs = [lanes | (iota ^ pb) for pb in range(16)]

                    def loads(qv):
                        bq = jnp.full((G,), 0, jnp.int32) + qv * 16
                        return bq, [
                            plsc.load_gather(tbl_v, [gbs[pb] | bq])
                            for pb in range(16)
                        ]

                    def stores(pbq, pvals):
                        for pb in range(16):
                            plsc.store_scatter(bufs[b], [sbs[pb] | pbq], pvals[pb])

                    bq0, vals0 = loads(jnp.int32(0))

                    def qbody(q, carry):
                        pbq = carry[0]
                        pvals = carry[1:]
                        nbq, nvals = loads(q)
                        stores(pbq, pvals)
                        return (nbq, *nvals)

                    fin = lax.fori_loop(1, 32, qbody, (bq0, *vals0))
                    stores(fin[0], fin[1:])
                    row0 = base + g * G
                    for r in range(G):
                        flat = pl.multiple_of((row0 + r) * D + p * HALF, 8)
                        pltpu.async_copy(
                            bufs[b].at[pl.ds(r * HALF, HALF)],
                            out_hbm.at[pl.ds(flat, HALF)],
                            ssems[b],
                        )
                return c2

            lax.fori_loop(0, NGROUP // 2, pair, 0)
            return carry

        lax.fori_loop(0, D // HALF, one_pass, 0)
        s_drain(0)
        s_drain(1)

    return k(ids_flat, table_halves)


def kernel(phoneme_ids, table):
    ids_flat = phoneme_ids.reshape(-1)
    table_halves = table.reshape(VOCAB, 2, HALF).transpose(1, 0, 2).reshape(-1)
    out = _sc_gather(ids_flat, table_halves)
    return out.reshape(phoneme_ids.shape[0], phoneme_ids.shape[1], D)
